# Initial kernel scaffold; baseline (speedup 1.0000x reference)
import jax
import jax.numpy as jnp
from jax.experimental import pallas as pl


def kernel(ego_feat_pairs, rel_feat_pairs, ego_idx, num_ego, Wq, Wk, Wb, gq, bq, gk, bk):
    raise NotImplementedError("write your pallas kernel here")


# speedup vs baseline: 4.3507x; 4.3507x over previous
import functools
import math

import jax
import jax.numpy as jnp
from jax import lax
from jax.experimental import pallas as pl
from jax.experimental.pallas import tpu as pltpu
from jax.experimental.pallas import tpu_sc as plsc

_E = 160000
_N = 10000
_D = 256
_H = 4
_DK = 64
_TEMP = 1.0 / math.sqrt(_DK)

_BE = 640
_GRID = _E // _BE

_NTILES = 32
_EPAD = 163840
_CHUNK = _EPAD // _NTILES
_BATCH = 128
_NB = _CHUNK // _BATCH
_SUB = 1024
_NSUB = _CHUNK // _SUB
_NVEC = _SUB // 16
_NPAD = 10240
_STRIPE = _NPAD // 16


def _dense_body(ego_ref, rel_ref, wq_ref, wk_ref, wb_ref, seg_ref,
                gq_ref, bq_ref, gk_ref, bk_ref, out_ref):
    eg = ego_ref[...]
    m = jnp.mean(eg, axis=1, keepdims=True)
    v = jnp.mean((eg - m) ** 2, axis=1, keepdims=True)
    egn = (eg - m) * lax.rsqrt(v + 1e-5) * gq_ref[...] + bq_ref[...]
    rl = rel_ref[...]
    m2 = jnp.mean(rl, axis=1, keepdims=True)
    v2 = jnp.mean((rl - m2) ** 2, axis=1, keepdims=True)
    rln = (rl - m2) * lax.rsqrt(v2 + 1e-5) * gk_ref[...] + bk_ref[...]
    q = jnp.dot(egn, wq_ref[...], preferred_element_type=jnp.float32)
    k = jnp.dot(rln, wk_ref[...], preferred_element_type=jnp.float32)
    qk = q * k
    logits = jnp.dot(qk, seg_ref[...], preferred_element_type=jnp.float32)
    logits = logits + jnp.dot(rln, wb_ref[...], preferred_element_type=jnp.float32)
    lane = lax.broadcasted_iota(jnp.int32, (_BE, 16), 1)
    out_ref[...] = jnp.where(lane < _H, jnp.exp(logits), 0.0)


def _dense(ego, rel, wq, wk, wb16, seg, gq, bq, gk, bk):
    full = lambda shape: pl.BlockSpec(shape, lambda i: (0, 0))
    return pl.pallas_call(
        _dense_body,
        grid=(_GRID,),
        in_specs=[
            pl.BlockSpec((_BE, _D), lambda i: (i, 0)),
            pl.BlockSpec((_BE, _D), lambda i: (i, 0)),
            full((_D, _D)), full((_D, _D)), full((_D, 16)), full((_D, 16)),
            full((1, _D)), full((1, _D)), full((1, _D)), full((1, _D)),
        ],
        out_specs=pl.BlockSpec((_BE, 16), lambda i: (i, 0)),
        out_shape=jax.ShapeDtypeStruct((_E, 16), jnp.float32),
    )(ego, rel, wq, wk, wb16, seg, gq, bq, gk, bk)


_MESH = plsc.VectorSubcoreMesh(core_axis_name="c", subcore_axis_name="s")
_SC_PARAMS = pltpu.CompilerParams(needs_layout_passes=False)


_DLEN = _NPAD * _H
_STRW = _DLEN // _NTILES


@functools.partial(
    pl.kernel,
    mesh=_MESH,
    out_type=jax.ShapeDtypeStruct((_NTILES, _DLEN), jnp.float32),
    compiler_params=_SC_PARAMS,
    scratch_types=[
        pltpu.VMEM((_SUB,), jnp.int32),
        pltpu.VMEM((_SUB * 16,), jnp.float32),
        pltpu.VMEM((_DLEN,), jnp.float32),
    ],
)
def _scatter(wf_hbm, idx_hbm, out_hbm, idx_v, w_v, acc_v):
    c = lax.axis_index("c")
    s = lax.axis_index("s")
    tid = s * 2 + c
    base = tid * _CHUNK

    def zero(i, carry):
        acc_v[pl.ds(i * 16, 16)] = jnp.zeros((16,), jnp.float32)
        return carry

    lax.fori_loop(0, _DLEN // 16, zero, 0)

    def sub(b, carry):
        off = base + b * _SUB
        pltpu.sync_copy(idx_hbm.at[pl.ds(off, _SUB)], idx_v)
        pltpu.sync_copy(wf_hbm.at[pl.ds(off * 16, _SUB * 16)], w_v)

        def vec(vi, carry2):
            idx16 = idx_v[pl.ds(vi * 16, 16)]
            row16 = lax.iota(jnp.int32, 16) + vi * 16
            for h in range(_H):
                wv = plsc.load_gather(w_v, [row16 * 16 + h])
                plsc.addupdate_scatter(acc_v, [idx16 * _H + h], wv)
            return carry2

        lax.fori_loop(0, _NVEC, vec, 0)
        return carry

    lax.fori_loop(0, _NSUB, sub, 0)
    pltpu.sync_copy(acc_v, out_hbm.at[tid])


@functools.partial(
    pl.kernel,
    mesh=_MESH,
    out_type=jax.ShapeDtypeStruct((_DLEN,), jnp.float32),
    compiler_params=_SC_PARAMS,
    scratch_types=[
        pltpu.VMEM((_STRW,), jnp.float32),
        pltpu.VMEM((_STRW,), jnp.float32),
    ],
)
def _reduce(part_hbm, out_hbm, tmp_v, acc_v):
    c = lax.axis_index("c")
    s = lax.axis_index("s")
    tid = s * 2 + c
    col = tid * _STRW

    def zero(i, carry):
        acc_v[pl.ds(i * 16, 16)] = jnp.zeros((16,), jnp.float32)
        return carry

    lax.fori_loop(0, _STRW // 16, zero, 0)

    def part(j, carry):
        pltpu.sync_copy(part_hbm.at[j, pl.ds(col, _STRW)], tmp_v)

        def add(i, carry2):
            sl = pl.ds(i * 16, 16)
            acc_v[sl] = acc_v[sl] + tmp_v[sl]
            return carry2

        lax.fori_loop(0, _STRW // 16, add, 0)
        return carry

    lax.fori_loop(0, _NTILES, part, 0)
    pltpu.sync_copy(acc_v, out_hbm.at[pl.ds(col, _STRW)])


@functools.partial(
    pl.kernel,
    mesh=_MESH,
    out_type=jax.ShapeDtypeStruct((_EPAD,), jnp.float32),
    compiler_params=_SC_PARAMS,
    scratch_types=[
        pltpu.VMEM((_DLEN,), jnp.float32),
        pltpu.VMEM((_SUB * 16,), jnp.float32),
        pltpu.VMEM((_SUB,), jnp.int32),
        pltpu.VMEM((_SUB,), jnp.float32),
    ],
)
def _gather(wf_hbm, idx_hbm, d_hbm, out_hbm, d_v, w_v, idx_v, out_v):
    c = lax.axis_index("c")
    s = lax.axis_index("s")
    tid = s * 2 + c
    base = tid * _CHUNK
    pltpu.sync_copy(d_hbm, d_v)

    def sub(b, carry):
        off = base + b * _SUB
        pltpu.sync_copy(idx_hbm.at[pl.ds(off, _SUB)], idx_v)
        pltpu.sync_copy(wf_hbm.at[pl.ds(off * 16, _SUB * 16)], w_v)

        def vec(vi, carry2):
            idx16 = idx_v[pl.ds(vi * 16, 16)]
            row16 = lax.iota(jnp.int32, 16) + vi * 16
            acc = jnp.zeros((16,), jnp.float32)
            for h in range(_H):
                wv = plsc.load_gather(w_v, [row16 * 16 + h])
                dv = plsc.load_gather(d_v, [idx16 * _H + h])
                acc = acc + wv / (dv + 1e-6)
            out_v[pl.ds(vi * 16, 16)] = acc * (1.0 / _H)
            return carry2

        lax.fori_loop(0, _NVEC, vec, 0)
        pltpu.sync_copy(out_v, out_hbm.at[pl.ds(off, _SUB)])
        return carry

    lax.fori_loop(0, _NSUB, sub, 0)


def kernel(ego_feat_pairs, rel_feat_pairs, ego_idx, num_ego,
           Wq, Wk, Wb, gq, bq, gk, bk):
    f32 = jnp.float32
    wb16 = jnp.concatenate([Wb.astype(f32), jnp.zeros((_D, 16 - _H), f32)], axis=1)
    d_iota = lax.broadcasted_iota(jnp.int32, (_D, 16), 0)
    h_iota = lax.broadcasted_iota(jnp.int32, (_D, 16), 1)
    seg = jnp.where((d_iota // _DK == h_iota) & (h_iota < _H), _TEMP, 0.0).astype(f32)

    w16 = _dense(ego_feat_pairs, rel_feat_pairs, Wq, Wk, wb16, seg,
                 gq.reshape(1, _D), bq.reshape(1, _D),
                 gk.reshape(1, _D), bk.reshape(1, _D))

    w16p = jnp.concatenate([w16, jnp.zeros((_EPAD - _E, 16), f32)], axis=0)
    idxp = jnp.concatenate(
        [ego_idx.astype(jnp.int32), jnp.zeros((_EPAD - _E,), jnp.int32)])
    wf = w16p.reshape(-1)

    partials = _scatter(wf, idxp)
    denf = _reduce(partials)
    alphap = _gather(wf, idxp, denf)
    return alphap[:_E, None]


# speedup vs baseline: 4.5796x; 1.0526x over previous
import functools
import math

import jax
import jax.numpy as jnp
from jax import lax
from jax.experimental import pallas as pl
from jax.experimental.pallas import tpu as pltpu
from jax.experimental.pallas import tpu_sc as plsc

_E = 160000
_N = 10000
_D = 256
_H = 4
_DK = 64
_TEMP = 1.0 / math.sqrt(_DK)

_BE = 640
_GRID = _E // _BE

_NTILES = 32
_EPAD = 163840
_CHUNK = _EPAD // _NTILES
_BATCH = 128
_NB = _CHUNK // _BATCH
_SUB = 1024
_NSUB = _CHUNK // _SUB
_NVEC = _SUB // 16
_NPAD = 10240
_STRIPE = _NPAD // 16


def _dense_body(ego_ref, rel_ref, wq_ref, wk_ref, wb_ref, seg_ref,
                gq_ref, bq_ref, gk_ref, bk_ref, out_ref):
    eg = ego_ref[...]
    m = jnp.mean(eg, axis=1, keepdims=True)
    v = jnp.mean((eg - m) ** 2, axis=1, keepdims=True)
    egn = (eg - m) * lax.rsqrt(v + 1e-5) * gq_ref[...] + bq_ref[...]
    rl = rel_ref[...]
    m2 = jnp.mean(rl, axis=1, keepdims=True)
    v2 = jnp.mean((rl - m2) ** 2, axis=1, keepdims=True)
    rln = (rl - m2) * lax.rsqrt(v2 + 1e-5) * gk_ref[...] + bk_ref[...]
    q = jnp.dot(egn, wq_ref[...], preferred_element_type=jnp.float32)
    k = jnp.dot(rln, wk_ref[...], preferred_element_type=jnp.float32)
    qk = q * k
    logits = jnp.dot(qk, seg_ref[...], preferred_element_type=jnp.float32)
    logits = logits + jnp.dot(rln, wb_ref[...], preferred_element_type=jnp.float32)
    lane = lax.broadcasted_iota(jnp.int32, (_BE, 16), 1)
    out_ref[...] = jnp.where(lane < _H, jnp.exp(logits), 0.0)


def _dense(ego, rel, wq, wk, wb16, seg, gq, bq, gk, bk):
    full = lambda shape: pl.BlockSpec(shape, lambda i: (0, 0))
    return pl.pallas_call(
        _dense_body,
        grid=(_GRID,),
        in_specs=[
            pl.BlockSpec((_BE, _D), lambda i: (i, 0)),
            pl.BlockSpec((_BE, _D), lambda i: (i, 0)),
            full((_D, _D)), full((_D, _D)), full((_D, 16)), full((_D, 16)),
            full((1, _D)), full((1, _D)), full((1, _D)), full((1, _D)),
        ],
        out_specs=pl.BlockSpec((_BE, 16), lambda i: (i, 0)),
        out_shape=jax.ShapeDtypeStruct((_E, 16), jnp.float32),
    )(ego, rel, wq, wk, wb16, seg, gq, bq, gk, bk)


_MESH = plsc.VectorSubcoreMesh(core_axis_name="c", subcore_axis_name="s")
_SC_PARAMS = pltpu.CompilerParams(needs_layout_passes=False)


_DLEN = _NPAD * _H
_STRW = _DLEN // _NTILES


@functools.partial(
    pl.kernel,
    mesh=_MESH,
    out_type=jax.ShapeDtypeStruct((_NTILES, _DLEN), jnp.float32),
    compiler_params=_SC_PARAMS,
    scratch_types=[
        pltpu.VMEM((_SUB,), jnp.int32),
        pltpu.VMEM((_SUB * 16,), jnp.float32),
        pltpu.VMEM((_DLEN,), jnp.float32),
    ],
)
def _scatter(wf_hbm, idx_hbm, out_hbm, idx_v, w_v, acc_v):
    c = lax.axis_index("c")
    s = lax.axis_index("s")
    tid = s * 2 + c
    base = tid * _CHUNK

    def zero(i, carry):
        acc_v[pl.ds(i * 16, 16)] = jnp.zeros((16,), jnp.float32)
        return carry

    lax.fori_loop(0, _DLEN // 16, zero, 0)

    def sub(b, carry):
        off = base + b * _SUB
        pltpu.sync_copy(idx_hbm.at[pl.ds(off, _SUB)], idx_v)
        pltpu.sync_copy(wf_hbm.at[pl.ds(off * 16, _SUB * 16)], w_v)

        def vec(vi, carry2):
            idx16 = idx_v[pl.ds(vi * 16, 16)]
            row16 = lax.iota(jnp.int32, 16) + vi * 16
            for h in range(_H):
                wv = plsc.load_gather(w_v, [row16 * 16 + h])
                plsc.addupdate_scatter(acc_v, [idx16 * _H + h], wv)
            return carry2

        lax.fori_loop(0, _NVEC, vec, 0)
        return carry

    lax.fori_loop(0, _NSUB, sub, 0)
    pltpu.sync_copy(acc_v, out_hbm.at[tid])


@functools.partial(
    pl.kernel,
    mesh=_MESH,
    out_type=jax.ShapeDtypeStruct((_DLEN,), jnp.float32),
    compiler_params=_SC_PARAMS,
    scratch_types=[
        pltpu.VMEM((_NTILES, _STRW), jnp.float32),
        pltpu.VMEM((_STRW,), jnp.float32),
    ],
)
def _reduce(part_hbm, out_hbm, tmp_v, acc_v):
    c = lax.axis_index("c")
    s = lax.axis_index("s")
    tid = s * 2 + c
    col = tid * _STRW
    pltpu.sync_copy(part_hbm.at[:, pl.ds(col, _STRW)], tmp_v)

    def add(i, carry):
        sl = pl.ds(i * 16, 16)
        v = tmp_v[0, sl]
        for j in range(1, _NTILES):
            v = v + tmp_v[j, sl]
        acc_v[sl] = v
        return carry

    lax.fori_loop(0, _STRW // 16, add, 0)
    pltpu.sync_copy(acc_v, out_hbm.at[pl.ds(col, _STRW)])


@functools.partial(
    pl.kernel,
    mesh=_MESH,
    out_type=jax.ShapeDtypeStruct((_EPAD,), jnp.float32),
    compiler_params=_SC_PARAMS,
    scratch_types=[
        pltpu.VMEM((_DLEN,), jnp.float32),
        pltpu.VMEM((_SUB * 16,), jnp.float32),
        pltpu.VMEM((_SUB,), jnp.int32),
        pltpu.VMEM((_SUB,), jnp.float32),
    ],
)
def _gather(wf_hbm, idx_hbm, d_hbm, out_hbm, d_v, w_v, idx_v, out_v):
    c = lax.axis_index("c")
    s = lax.axis_index("s")
    tid = s * 2 + c
    base = tid * _CHUNK
    pltpu.sync_copy(d_hbm, d_v)

    def sub(b, carry):
        off = base + b * _SUB
        pltpu.sync_copy(idx_hbm.at[pl.ds(off, _SUB)], idx_v)
        pltpu.sync_copy(wf_hbm.at[pl.ds(off * 16, _SUB * 16)], w_v)

        def vec(vi, carry2):
            idx16 = idx_v[pl.ds(vi * 16, 16)]
            row16 = lax.iota(jnp.int32, 16) + vi * 16
            acc = jnp.zeros((16,), jnp.float32)
            for h in range(_H):
                wv = plsc.load_gather(w_v, [row16 * 16 + h])
                dv = plsc.load_gather(d_v, [idx16 * _H + h])
                acc = acc + wv / (dv + 1e-6)
            out_v[pl.ds(vi * 16, 16)] = acc * (1.0 / _H)
            return carry2

        lax.fori_loop(0, _NVEC, vec, 0)
        pltpu.sync_copy(out_v, out_hbm.at[pl.ds(off, _SUB)])
        return carry

    lax.fori_loop(0, _NSUB, sub, 0)


def kernel(ego_feat_pairs, rel_feat_pairs, ego_idx, num_ego,
           Wq, Wk, Wb, gq, bq, gk, bk):
    f32 = jnp.float32
    wb16 = jnp.concatenate([Wb.astype(f32), jnp.zeros((_D, 16 - _H), f32)], axis=1)
    d_iota = lax.broadcasted_iota(jnp.int32, (_D, 16), 0)
    h_iota = lax.broadcasted_iota(jnp.int32, (_D, 16), 1)
    seg = jnp.where((d_iota // _DK == h_iota) & (h_iota < _H), _TEMP, 0.0).astype(f32)

    w16 = _dense(ego_feat_pairs, rel_feat_pairs, Wq, Wk, wb16, seg,
                 gq.reshape(1, _D), bq.reshape(1, _D),
                 gk.reshape(1, _D), bk.reshape(1, _D))

    w16p = jnp.concatenate([w16, jnp.zeros((_EPAD - _E, 16), f32)], axis=0)
    idxp = jnp.concatenate(
        [ego_idx.astype(jnp.int32), jnp.zeros((_EPAD - _E,), jnp.int32)])
    wf = w16p.reshape(-1)

    partials = _scatter(wf, idxp)
    denf = _reduce(partials)
    alphap = _gather(wf, idxp, denf)
    return alphap[:_E, None]
